# baseline (device time: 18675 ns/iter reference)
import jax
import jax.numpy as jnp
from jax import lax
from jax.experimental import pallas as pl
from jax.experimental.pallas import tpu as pltpu

BLK_ROWS = 512


def kernel(x, dy, gamma):
    m, d = x.shape
    half = m // 2
    grid = half // BLK_ROWS

    def body(off_ref, x_ref, dy_ref, out_ref, acc_ref, comm_ref,
             send_sems, recv_sems):
        step = pl.program_id(0)
        my_x = lax.axis_index("x")
        my_y = lax.axis_index("y")
        peers = [(my_x, 1 - my_y), (1 - my_x, my_y), (1 - my_x, 1 - my_y)]

        @pl.when(step == 0)
        def _():
            barrier = pltpu.get_barrier_semaphore()
            for p in peers:
                pl.semaphore_signal(
                    barrier, inc=1, device_id=p,
                    device_id_type=pl.DeviceIdType.MESH,
                )
            pl.semaphore_wait(barrier, 3)

        xb = x_ref[...]
        dyb = dy_ref[...]
        ones_col = jnp.ones((d, 1), jnp.float32)
        inv_d = 1.0 / d
        mu = jnp.dot(xb, ones_col, preferred_element_type=jnp.float32) * inv_d
        ex2 = (
            jnp.dot(xb * xb, ones_col, preferred_element_type=jnp.float32)
            * inv_d
        )
        rstd = lax.rsqrt(ex2 - mu * mu + 1e-5)
        dgamma = jnp.sum(dyb * (xb * rstd - mu * rstd), axis=0, keepdims=True)
        dbeta = jnp.sum(dyb, axis=0, keepdims=True)
        acc_ref[step] = jnp.concatenate([dgamma, dbeta], axis=0)
        rdmas = []
        for slot, p in enumerate(peers):
            r = pltpu.make_async_remote_copy(
                src_ref=acc_ref.at[step],
                dst_ref=comm_ref.at[step, slot],
                send_sem=send_sems.at[step, slot],
                recv_sem=recv_sems.at[step, slot],
                device_id=p,
                device_id_type=pl.DeviceIdType.MESH,
            )
            r.start()
            rdmas.append(r)

        del rdmas

        @pl.when(step == grid - 1)
        def _():
            for ph in range(grid):
                for slot, p in enumerate(peers):
                    dsc = pltpu.make_async_remote_copy(
                        src_ref=acc_ref.at[ph],
                        dst_ref=comm_ref.at[ph, slot],
                        send_sem=send_sems.at[ph, slot],
                        recv_sem=recv_sems.at[ph, slot],
                        device_id=p,
                        device_id_type=pl.DeviceIdType.MESH,
                    )
                    dsc.wait_send()
                    dsc.wait_recv()
            total = acc_ref[0]
            for ph in range(1, grid):
                total += acc_ref[ph]
            for ph in range(grid):
                for slot in range(3):
                    total += comm_ref[ph, slot]
            out_ref[...] = total

    off = (lax.axis_index("x") * grid).astype(jnp.int32).reshape((1,))
    grid_spec = pltpu.PrefetchScalarGridSpec(
        num_scalar_prefetch=1,
        grid=(grid,),
        in_specs=[
            pl.BlockSpec((BLK_ROWS, d), lambda i, off: (off[0] + i, 0)),
            pl.BlockSpec((BLK_ROWS, d), lambda i, off: (off[0] + i, 0)),
        ],
        out_specs=pl.BlockSpec((2, d), lambda i, off: (0, 0)),
        scratch_shapes=[
            pltpu.VMEM((grid, 2, d), jnp.float32),
            pltpu.VMEM((grid, 3, 2, d), jnp.float32),
            pltpu.SemaphoreType.DMA((grid, 3)),
            pltpu.SemaphoreType.DMA((grid, 3)),
        ],
    )
    return pl.pallas_call(
        body,
        grid_spec=grid_spec,
        out_shape=jax.ShapeDtypeStruct((2, d), jnp.float32),
        compiler_params=pltpu.CompilerParams(collective_id=0),
    )(off, x, dy)


# device time: 16089 ns/iter; 1.1607x vs baseline; 1.1607x over previous
import jax
import jax.numpy as jnp
from jax import lax
from jax.experimental import pallas as pl
from jax.experimental.pallas import tpu as pltpu

CH_ROWS = 128


def kernel(x, dy, gamma):
    m, d = x.shape
    half = m // 2
    n_chunks = half // CH_ROWS

    def body(x_hbm, dy_hbm, out_ref, xv, dyv, acc_ref, comm_ref,
             x_sems, dy_sems, send_sems, recv_sems):
        my_x = lax.axis_index("x")
        my_y = lax.axis_index("y")
        peers = [(my_x, 1 - my_y), (1 - my_x, my_y), (1 - my_x, 1 - my_y)]

        barrier = pltpu.get_barrier_semaphore()
        for p in peers:
            pl.semaphore_signal(
                barrier, inc=1, device_id=p,
                device_id_type=pl.DeviceIdType.MESH,
            )
        pl.semaphore_wait(barrier, 3)

        row0 = my_x * half
        copies = []
        for c in range(n_chunks):
            rows = pl.ds(row0 + c * CH_ROWS, CH_ROWS)
            cx = pltpu.make_async_copy(x_hbm.at[rows, :], xv.at[c],
                                       x_sems.at[c])
            cd = pltpu.make_async_copy(dy_hbm.at[rows, :], dyv.at[c],
                                       dy_sems.at[c])
            cx.start()
            cd.start()
            copies.append((cx, cd))

        dg = jnp.zeros((1, d), jnp.float32)
        db = jnp.zeros((1, d), jnp.float32)
        for c in range(n_chunks):
            cx, cd = copies[c]
            cx.wait()
            cd.wait()
            xb = xv[c]
            dyb = dyv[c]
            mu = jnp.mean(xb, axis=1, keepdims=True)
            ex2 = jnp.mean(xb * xb, axis=1, keepdims=True)
            rstd = lax.rsqrt(ex2 - mu * mu + 1e-5)
            dg += jnp.sum(dyb * (xb * rstd - mu * rstd), axis=0,
                          keepdims=True)
            db += jnp.sum(dyb, axis=0, keepdims=True)
        acc_ref[...] = jnp.concatenate([dg, db], axis=0)

        rdmas = []
        for slot, p in enumerate(peers):
            r = pltpu.make_async_remote_copy(
                src_ref=acc_ref,
                dst_ref=comm_ref.at[slot],
                send_sem=send_sems.at[slot],
                recv_sem=recv_sems.at[slot],
                device_id=p,
                device_id_type=pl.DeviceIdType.MESH,
            )
            r.start()
            rdmas.append(r)
        for r in rdmas:
            r.wait()
        out_ref[...] = acc_ref[...] + comm_ref[0] + comm_ref[1] + comm_ref[2]

    return pl.pallas_call(
        body,
        out_shape=jax.ShapeDtypeStruct((2, d), jnp.float32),
        in_specs=[
            pl.BlockSpec(memory_space=pl.ANY),
            pl.BlockSpec(memory_space=pl.ANY),
        ],
        out_specs=pl.BlockSpec(memory_space=pltpu.VMEM),
        scratch_shapes=[
            pltpu.VMEM((n_chunks, CH_ROWS, d), jnp.float32),
            pltpu.VMEM((n_chunks, CH_ROWS, d), jnp.float32),
            pltpu.VMEM((2, d), jnp.float32),
            pltpu.VMEM((3, 2, d), jnp.float32),
            pltpu.SemaphoreType.DMA((n_chunks,)),
            pltpu.SemaphoreType.DMA((n_chunks,)),
            pltpu.SemaphoreType.DMA((3,)),
            pltpu.SemaphoreType.DMA((3,)),
        ],
        compiler_params=pltpu.CompilerParams(collective_id=0),
    )(x, dy)


# device time: 13432 ns/iter; 1.3903x vs baseline; 1.1978x over previous
import jax
import jax.numpy as jnp
from jax import lax
from jax.experimental import pallas as pl
from jax.experimental.pallas import tpu as pltpu

CH_ROWS = 128


def kernel(x, dy, gamma):
    m, d = x.shape
    half = m // 2
    n_chunks = half // CH_ROWS

    def body(x_hbm, dy_hbm, out_ref, xv, dyv, acc_ref, comm_ref, pad_ref,
             x_sems, dy_sems, send_sems, recv_sems, out_sem):
        my_x = lax.axis_index("x")
        my_y = lax.axis_index("y")
        peers = [(my_x, 1 - my_y), (1 - my_x, my_y), (1 - my_x, 1 - my_y)]

        barrier = pltpu.get_barrier_semaphore()
        for p in peers:
            pl.semaphore_signal(
                barrier, inc=1, device_id=p,
                device_id_type=pl.DeviceIdType.MESH,
            )

        pad_ref[0, 0:8, 0:128] = jnp.zeros((8, 128), jnp.float32)

        row0 = my_x * half
        copies = []
        for c in range(n_chunks):
            rows = pl.ds(row0 + c * CH_ROWS, CH_ROWS)
            cx = pltpu.make_async_copy(x_hbm.at[rows, :], xv.at[c],
                                       x_sems.at[c])
            cd = pltpu.make_async_copy(dy_hbm.at[rows, :], dyv.at[c],
                                       dy_sems.at[c])
            cx.start()
            cd.start()
            copies.append((cx, cd))

        dg = jnp.zeros((1, d), jnp.float32)
        db = jnp.zeros((1, d), jnp.float32)
        for c in range(n_chunks):
            cx, cd = copies[c]
            cx.wait()
            cd.wait()
            xb = xv[c]
            dyb = dyv[c]
            mu = jnp.mean(xb, axis=1, keepdims=True)
            ex2 = jnp.mean(xb * xb, axis=1, keepdims=True)
            rstd = lax.rsqrt(ex2 - mu * mu + 1e-5)
            dg += jnp.sum(dyb * (xb * rstd - mu * rstd), axis=0,
                          keepdims=True)
            db += jnp.sum(dyb, axis=0, keepdims=True)
        acc_ref[...] = jnp.concatenate([dg, db], axis=0)

        pl.semaphore_wait(barrier, 3)
        rdmas = []
        for slot, p in enumerate(peers):
            r = pltpu.make_async_remote_copy(
                src_ref=acc_ref,
                dst_ref=comm_ref.at[slot],
                send_sem=send_sems.at[slot],
                recv_sem=recv_sems.at[slot],
                device_id=p,
                device_id_type=pl.DeviceIdType.MESH,
            )
            r.start()
            rdmas.append(r)
        for r in rdmas:
            r.wait()
        acc_ref[...] += comm_ref[0] + comm_ref[1] + comm_ref[2]
        out_copy = pltpu.make_async_copy(acc_ref, out_ref, out_sem)
        out_copy.start()
        out_copy.wait()

    return pl.pallas_call(
        body,
        out_shape=jax.ShapeDtypeStruct((2, d), jnp.float32),
        in_specs=[
            pl.BlockSpec(memory_space=pltpu.MemorySpace.HBM),
            pl.BlockSpec(memory_space=pltpu.MemorySpace.HBM),
        ],
        out_specs=pl.BlockSpec(memory_space=pltpu.MemorySpace.HBM),
        scratch_shapes=[
            pltpu.VMEM((n_chunks, CH_ROWS, d), jnp.float32),
            pltpu.VMEM((n_chunks, CH_ROWS, d), jnp.float32),
            pltpu.VMEM((2, d), jnp.float32),
            pltpu.VMEM((3, 2, d), jnp.float32),
            pltpu.VMEM((10, 1024, 1024), jnp.float32),
            pltpu.SemaphoreType.DMA((n_chunks,)),
            pltpu.SemaphoreType.DMA((n_chunks,)),
            pltpu.SemaphoreType.DMA((3,)),
            pltpu.SemaphoreType.DMA((3,)),
            pltpu.SemaphoreType.DMA,
        ],
        compiler_params=pltpu.CompilerParams(
            collective_id=0, vmem_limit_bytes=63 * 1024 * 1024
        ),
    )(x, dy)
